# baseline (speedup 1.0000x reference)
import functools

import jax
import jax.numpy as jnp
from jax import lax
from jax.experimental import pallas as pl
from jax.experimental.pallas import tpu as pltpu
from jax.experimental.pallas import tpu_sc as plsc

_N = 1048576
_NC, _NS, _L = 2, 16, 16
_NW = _NC * _NS
_CHUNK = _N // _NW
_NPIECE = 8
_P = _CHUNK // _NPIECE
_ALPHA = 0.1

_mesh = plsc.VectorSubcoreMesh(core_axis_name="c", subcore_axis_name="s")


@functools.partial(
    pl.kernel,
    mesh=_mesh,
    compiler_params=pltpu.CompilerParams(
        needs_layout_passes=False,
        disable_bounds_checks=True,
        skip_device_barrier=True,
    ),
    out_type=jax.ShapeDtypeStruct((_NW, _L), jnp.float32),
    scratch_types=[
        pltpu.VMEM((2, _P), jnp.float32),
        pltpu.VMEM((2, _P), jnp.int32),
        pltpu.VMEM((100,), jnp.float32),
        pltpu.VMEM((_L,), jnp.float32),
        pltpu.SemaphoreType.DMA,
        pltpu.SemaphoreType.DMA,
        pltpu.SemaphoreType.DMA,
        pltpu.SemaphoreType.DMA,
        pltpu.SemaphoreType.DMA,
    ],
)
def _pinball_partials(s_hbm, y_hbm, t_hbm, out_hbm,
                      s_v, y_v, t_v, o_v, sem_s0, sem_s1, sem_y0, sem_y1,
                      sem_t):
    wid = lax.axis_index("s") * _NC + lax.axis_index("c")
    base = wid * _CHUNK
    sem_s = (sem_s0, sem_s1)
    sem_y = (sem_y0, sem_y1)

    cp_t = pltpu.async_copy(t_hbm, t_v, sem_t)

    def start(k):
        b = k % 2
        cs = pltpu.async_copy(
            s_hbm.at[pl.ds(base + k * _P, _P)], s_v.at[b], sem_s[b])
        cy = pltpu.async_copy(
            y_hbm.at[pl.ds(base + k * _P, _P)], y_v.at[b], sem_y[b])
        return cs, cy

    pend = start(0)
    cp_t.wait()

    z = jnp.zeros((_L,), jnp.float32)
    accs = (z, z)
    for k in range(_NPIECE):
        b = k % 2
        cs, cy = pend
        if k + 1 < _NPIECE:
            pend = start(k + 1)
        cs.wait()
        cy.wait()

        def step(i, accs, _b=b):
            a0, a1 = accs
            s0 = s_v[_b, pl.ds(i, _L)]
            idx0 = y_v[_b, pl.ds(i, _L)]
            s1 = s_v[_b, pl.ds(i + _L, _L)]
            idx1 = y_v[_b, pl.ds(i + _L, _L)]
            d0 = plsc.load_gather(t_v, [idx0]) - s0
            d1 = plsc.load_gather(t_v, [idx1]) - s1
            a0 = a0 + jnp.maximum((1.0 - _ALPHA) * d0, -_ALPHA * d0)
            a1 = a1 + jnp.maximum((1.0 - _ALPHA) * d1, -_ALPHA * d1)
            return a0, a1

        accs = plsc.parallel_loop(0, _P, 2 * _L, unroll=8, carry=accs)(step)

    o_v[...] = accs[0] + accs[1]
    pltpu.sync_copy(o_v, out_hbm.at[wid])


def kernel(S, Y, y_pred):
    partials = _pinball_partials(S, Y.astype(jnp.int32), y_pred)
    return jnp.sum(partials) / _N


# speedup vs baseline: 1.0871x; 1.0871x over previous
import functools

import jax
import jax.numpy as jnp
from jax import lax
from jax.experimental import pallas as pl
from jax.experimental.pallas import tpu as pltpu
from jax.experimental.pallas import tpu_sc as plsc

_N = 1048576
_NC, _NS, _L = 2, 16, 16
_NW = _NC * _NS
_CHUNK = _N // _NW
_NPIECE = 2
_P = _CHUNK // _NPIECE
_ALPHA = 0.1

_mesh = plsc.VectorSubcoreMesh(core_axis_name="c", subcore_axis_name="s")


@functools.partial(
    pl.kernel,
    mesh=_mesh,
    compiler_params=pltpu.CompilerParams(
        needs_layout_passes=False,
        disable_bounds_checks=True,
        skip_device_barrier=True,
    ),
    out_type=jax.ShapeDtypeStruct((_NW, _L), jnp.float32),
    scratch_types=[
        pltpu.VMEM((2, _P), jnp.float32),
        pltpu.VMEM((2, _P), jnp.int32),
        pltpu.VMEM((100,), jnp.float32),
        pltpu.VMEM((_L,), jnp.float32),
        pltpu.SemaphoreType.DMA,
        pltpu.SemaphoreType.DMA,
        pltpu.SemaphoreType.DMA,
        pltpu.SemaphoreType.DMA,
        pltpu.SemaphoreType.DMA,
    ],
)
def _pinball_partials(s_hbm, y_hbm, t_hbm, out_hbm,
                      s_v, y_v, t_v, o_v, sem_s0, sem_s1, sem_y0, sem_y1,
                      sem_t):
    wid = lax.axis_index("s") * _NC + lax.axis_index("c")
    base = wid * _CHUNK
    sem_s = (sem_s0, sem_s1)
    sem_y = (sem_y0, sem_y1)

    cp_t = pltpu.async_copy(t_hbm, t_v, sem_t)

    def start(k):
        b = k % 2
        cs = pltpu.async_copy(
            s_hbm.at[pl.ds(base + k * _P, _P)], s_v.at[b], sem_s[b])
        cy = pltpu.async_copy(
            y_hbm.at[pl.ds(base + k * _P, _P)], y_v.at[b], sem_y[b])
        return cs, cy

    pend = start(0)
    cp_t.wait()

    z = jnp.zeros((_L,), jnp.float32)
    accs = (z, z)
    for k in range(_NPIECE):
        b = k % 2
        cs, cy = pend
        if k + 1 < _NPIECE:
            pend = start(k + 1)
        cs.wait()
        cy.wait()

        def step(i, accs, _b=b):
            a0, a1 = accs
            s0 = s_v[_b, pl.ds(i, _L)]
            idx0 = y_v[_b, pl.ds(i, _L)]
            s1 = s_v[_b, pl.ds(i + _L, _L)]
            idx1 = y_v[_b, pl.ds(i + _L, _L)]
            d0 = plsc.load_gather(t_v, [idx0]) - s0
            d1 = plsc.load_gather(t_v, [idx1]) - s1
            a0 = a0 + jnp.maximum((1.0 - _ALPHA) * d0, -_ALPHA * d0)
            a1 = a1 + jnp.maximum((1.0 - _ALPHA) * d1, -_ALPHA * d1)
            return a0, a1

        accs = plsc.parallel_loop(0, _P, 2 * _L, unroll=8, carry=accs)(step)

    o_v[...] = accs[0] + accs[1]
    pltpu.sync_copy(o_v, out_hbm.at[wid])


def kernel(S, Y, y_pred):
    partials = _pinball_partials(S, Y.astype(jnp.int32), y_pred)
    return jnp.sum(partials) / _N
